# baseline (speedup 1.0000x reference)
import functools

import jax
import jax.numpy as jnp
from jax import lax
from jax.experimental import pallas as pl
from jax.experimental.pallas import tpu as pltpu
from jax.experimental.pallas import tpu_sc as plsc

A = 8
NWORKERS = 32
SLOTS = 8


def _tc_body(h_ref, pi_ref, act_ref,
             kW1_ref, kb1_ref, kW2_ref, kb2_ref,
             qW1_ref, qb1_ref, qW2_ref, qb2_ref,
             zf_ref, w_ref):
    R, _ = h_ref.shape
    NA = pi_ref.shape[1]
    E = R // A

    hb = h_ref[...]
    K = jnp.dot(jnp.tanh(jnp.dot(hb, kW1_ref[...], preferred_element_type=jnp.float32)
                         + kb1_ref[...]),
                kW2_ref[...], preferred_element_type=jnp.float32) + kb2_ref[...]
    Q = jnp.dot(jnp.tanh(jnp.dot(hb, qW1_ref[...], preferred_element_type=jnp.float32)
                         + qb1_ref[...]),
                qW2_ref[...], preferred_element_type=jnp.float32) + qb2_ref[...]

    sfull = jax.lax.dot_general(Q, K, (((1,), (1,)), ((), ())),
                                preferred_element_type=jnp.float32)
    rr = jax.lax.broadcasted_iota(jnp.int32, (R, R), 0) // A
    qq = jax.lax.broadcasted_iota(jnp.int32, (R, R), 1) // A
    sfull = jnp.where(rr == qq, sfull, 0.0)
    j8a = jax.lax.broadcasted_iota(jnp.int32, (R, A), 0) % A
    j8b = jax.lax.broadcasted_iota(jnp.int32, (R, A), 1)
    fold = (j8a == j8b).astype(jnp.float32)
    S2 = jnp.dot(sfull, fold, preferred_element_type=jnp.float32)
    w2 = jax.nn.sigmoid(S2 * 0.125)
    w_ref[...] = w2

    F = A * NA
    xa = jax.lax.broadcasted_iota(jnp.int32, (A, F), 0)
    xb = jax.lax.broadcasted_iota(jnp.int32, (A, F), 1) // NA
    xexp = (xa == xb).astype(jnp.float32)
    wfl = jnp.dot(w2, xexp, preferred_element_type=jnp.float32)
    pib = jnp.broadcast_to(pi_ref[...].reshape(E, 1, A, NA),
                           (E, A, A, NA)).reshape(R, F)
    actb = jnp.broadcast_to(act_ref[...].reshape(E, 1, A, NA),
                            (E, A, A, NA)).reshape(R, F)
    z = wfl * actb + (1.0 - wfl) * pib
    ca = jax.lax.broadcasted_iota(jnp.int32, (F, F), 0) % NA
    cb = jax.lax.broadcasted_iota(jnp.int32, (F, F), 1) % NA
    tsum = (ca == cb).astype(jnp.float32)
    ssum = jnp.dot(z, tsum, preferred_element_type=jnp.float32)
    zf_ref[...] = (pib + ssum - z) * 0.125


def _tc_stage(h, policies, actions, kW1, kb1r, kW2, kb2r, qW1, qb1r, qW2, qb2r,
              ns, base):
    D = h.shape[1]
    NA = policies.shape[1]
    F = A * NA
    HID = kW1.shape[1]
    OUT = kW2.shape[1]
    E = 32
    R = E * A
    grid = ns // R
    nb = base // R

    node_spec = lambda w: pl.BlockSpec((R, w), lambda i: (i + nb, 0))
    full_spec = lambda a, b: pl.BlockSpec((a, b), lambda i: (0, 0))

    return pl.pallas_call(
        _tc_body,
        grid=(grid,),
        in_specs=[
            node_spec(D), node_spec(NA), node_spec(NA),
            full_spec(D, HID), full_spec(1, HID),
            full_spec(HID, OUT), full_spec(1, OUT),
            full_spec(D, HID), full_spec(1, HID),
            full_spec(HID, OUT), full_spec(1, OUT),
        ],
        out_specs=[
            pl.BlockSpec((R, F), lambda i: (i, 0)),
            pl.BlockSpec((R, A), lambda i: (i, 0)),
        ],
        out_shape=[
            jax.ShapeDtypeStruct((ns, F), jnp.float32),
            jax.ShapeDtypeStruct((ns, A), jnp.float32),
        ],
        compiler_params=pltpu.CompilerParams(
            dimension_semantics=("arbitrary",)),
    )(h, policies, actions, kW1, kb1r, kW2, kb2r, qW1, qb1r, qW2, qb2r)


def _sc_assemble_body(eperw, obs_base, obs_hbm, zf_hbm, out_hbm,
                      obs_v, zf_v, zsh_v, sem_in, sem_out):
    D = obs_hbm.shape[1]
    NA = zf_hbm.shape[1] // A
    wid = lax.axis_index("s") * 2 + lax.axis_index("c")
    env0 = wid * eperw

    def in_copies(e, s):
        return (pltpu.make_async_copy(obs_hbm.at[pl.ds((obs_base + e) * A, A)],
                                      obs_v.at[s], sem_in.at[s]),
                pltpu.make_async_copy(zf_hbm.at[pl.ds(e * A, A)],
                                      zf_v.at[s], sem_in.at[s]))

    def out_copies(e, s):
        cps = [pltpu.make_async_copy(
                   obs_v.at[s],
                   out_hbm.at[e * A + i, :, pl.ds(0, D)],
                   sem_out.at[s]) for i in range(A)]
        cps.append(pltpu.make_async_copy(
            zsh_v.at[s],
            out_hbm.at[pl.ds(e * A, A), :, pl.ds(D, NA)],
            sem_out.at[s]))
        return cps

    def start_in(e, s):
        for c in in_copies(e, s):
            c.start()

    def step(k, _):
        s = lax.rem(k, SLOTS)
        sn = lax.rem(k + 1, SLOTS)
        @pl.when(k >= SLOTS - 1)
        def _():
            for c in out_copies(env0, sn):
                c.wait()
        @pl.when(k + 1 < eperw)
        def _():
            start_in(env0 + k + 1, sn)
        for c in in_copies(env0, s):
            c.wait()
        for i in range(A):
            for j in range(A):
                zsh_v[s, i, j, :] = zf_v[s, i, pl.ds(j * NA, NA)]
        for c in out_copies(env0 + k, s):
            c.start()
        return 0

    start_in(env0, 0)
    lax.fori_loop(0, eperw, step, 0)
    for s in range(1, SLOTS):
        for c in out_copies(env0, s):
            c.wait()


def _sc_stage(obs_proc, zf, env_base):
    D = obs_proc.shape[1]
    ns = zf.shape[0]
    NA = zf.shape[1] // A
    eperw = ns // (A * NWORKERS)
    mesh = plsc.VectorSubcoreMesh(core_axis_name="c", subcore_axis_name="s")
    fn = functools.partial(
        pl.kernel,
        functools.partial(_sc_assemble_body, eperw, env_base),
        out_type=jax.ShapeDtypeStruct((ns, A, D + NA), jnp.float32),
        mesh=mesh,
        scratch_types=[
            pltpu.VMEM((SLOTS, A, D), jnp.float32),
            pltpu.VMEM((SLOTS, A, A * NA), jnp.float32),
            pltpu.VMEM((SLOTS, A, A, NA), jnp.float32),
            pltpu.SemaphoreType.DMA((SLOTS,)),
            pltpu.SemaphoreType.DMA((SLOTS,)),
        ],
    )()
    return fn(obs_proc, zf)


def kernel(h, policies, actions, obs_proc, edge_index,
           kW1, kb1, kW2, kb2, qW1, qb1, qW2, qb2):
    N, D = h.shape
    NA = policies.shape[1]
    HID = kW1.shape[1]
    OUT = kW2.shape[1]

    kb1r = kb1.reshape(1, HID); kb2r = kb2.reshape(1, OUT)
    qb1r = qb1.reshape(1, HID); qb2r = qb2.reshape(1, OUT)

    SLABS = 1
    ns = N // SLABS
    outs, ws = [], []
    for s in range(SLABS):
        zf, w2 = _tc_stage(h, policies, actions,
                           kW1, kb1r, kW2, kb2r, qW1, qb1r, qW2, qb2r,
                           ns, s * ns)
        outs.append(_sc_stage(obs_proc, zf, s * ns // A))
        ws.append(w2)
    out_final = jnp.concatenate(outs, axis=0)
    w_final = jnp.concatenate(ws, axis=0)
    return out_final, w_final.reshape(N, A, 1)


# speedup vs baseline: 1.1111x; 1.1111x over previous
import functools

import jax
import jax.numpy as jnp
from jax import lax
from jax.experimental import pallas as pl
from jax.experimental.pallas import tpu as pltpu
from jax.experimental.pallas import tpu_sc as plsc

A = 8
NWORKERS = 32
SLOTS = 8


def _tc_body(h_ref, pi_ref, act_ref,
             kW1_ref, kb1_ref, kW2_ref, kb2_ref,
             qW1_ref, qb1_ref, qW2_ref, qb2_ref,
             zf_ref, w_ref):
    R, _ = h_ref.shape
    NA = pi_ref.shape[1]
    E = R // A

    hb = h_ref[...]
    K = jnp.dot(jnp.tanh(jnp.dot(hb, kW1_ref[...], preferred_element_type=jnp.float32)
                         + kb1_ref[...]),
                kW2_ref[...], preferred_element_type=jnp.float32) + kb2_ref[...]
    Q = jnp.dot(jnp.tanh(jnp.dot(hb, qW1_ref[...], preferred_element_type=jnp.float32)
                         + qb1_ref[...]),
                qW2_ref[...], preferred_element_type=jnp.float32) + qb2_ref[...]

    sfull = jax.lax.dot_general(Q, K, (((1,), (1,)), ((), ())),
                                preferred_element_type=jnp.float32)
    rr = jax.lax.broadcasted_iota(jnp.int32, (R, R), 0) // A
    qq = jax.lax.broadcasted_iota(jnp.int32, (R, R), 1) // A
    sfull = jnp.where(rr == qq, sfull, 0.0)
    j8a = jax.lax.broadcasted_iota(jnp.int32, (R, A), 0) % A
    j8b = jax.lax.broadcasted_iota(jnp.int32, (R, A), 1)
    fold = (j8a == j8b).astype(jnp.float32)
    S2 = jnp.dot(sfull, fold, preferred_element_type=jnp.float32)
    w2 = jax.nn.sigmoid(S2 * 0.125)
    w_ref[...] = w2

    F = A * NA
    xa = jax.lax.broadcasted_iota(jnp.int32, (A, F), 0)
    xb = jax.lax.broadcasted_iota(jnp.int32, (A, F), 1) // NA
    xexp = (xa == xb).astype(jnp.float32)
    wfl = jnp.dot(w2, xexp, preferred_element_type=jnp.float32)
    pib = jnp.broadcast_to(pi_ref[...].reshape(E, 1, A, NA),
                           (E, A, A, NA)).reshape(R, F)
    actb = jnp.broadcast_to(act_ref[...].reshape(E, 1, A, NA),
                            (E, A, A, NA)).reshape(R, F)
    z = wfl * actb + (1.0 - wfl) * pib
    ca = jax.lax.broadcasted_iota(jnp.int32, (F, F), 0) % NA
    cb = jax.lax.broadcasted_iota(jnp.int32, (F, F), 1) % NA
    tsum = (ca == cb).astype(jnp.float32)
    ssum = jnp.dot(z, tsum, preferred_element_type=jnp.float32)
    zf_ref[...] = (pib + ssum - z) * 0.125


def _tc_stage(h, policies, actions, kW1, kb1r, kW2, kb2r, qW1, qb1r, qW2, qb2r,
              ns, base):
    D = h.shape[1]
    NA = policies.shape[1]
    F = A * NA
    HID = kW1.shape[1]
    OUT = kW2.shape[1]
    E = 64
    R = E * A
    grid = ns // R
    nb = base // R

    node_spec = lambda w: pl.BlockSpec((R, w), lambda i: (i + nb, 0))
    full_spec = lambda a, b: pl.BlockSpec((a, b), lambda i: (0, 0))

    return pl.pallas_call(
        _tc_body,
        grid=(grid,),
        in_specs=[
            node_spec(D), node_spec(NA), node_spec(NA),
            full_spec(D, HID), full_spec(1, HID),
            full_spec(HID, OUT), full_spec(1, OUT),
            full_spec(D, HID), full_spec(1, HID),
            full_spec(HID, OUT), full_spec(1, OUT),
        ],
        out_specs=[
            pl.BlockSpec((R, F), lambda i: (i, 0)),
            pl.BlockSpec((R, A), lambda i: (i, 0)),
        ],
        out_shape=[
            jax.ShapeDtypeStruct((ns, F), jnp.float32),
            jax.ShapeDtypeStruct((ns, A), jnp.float32),
        ],
        compiler_params=pltpu.CompilerParams(
            dimension_semantics=("arbitrary",)),
    )(h, policies, actions, kW1, kb1r, kW2, kb2r, qW1, qb1r, qW2, qb2r)


def _sc_assemble_body(eperw, obs_base, obs_hbm, zf_hbm, out_hbm,
                      obs_v, zf_v, zsh_v, sem_in, sem_out):
    D = obs_hbm.shape[1]
    NA = zf_hbm.shape[1] // A
    wid = lax.axis_index("s") * 2 + lax.axis_index("c")
    env0 = wid * eperw

    def in_copies(e, s):
        return (pltpu.make_async_copy(obs_hbm.at[pl.ds((obs_base + e) * A, A)],
                                      obs_v.at[s], sem_in.at[s]),
                pltpu.make_async_copy(zf_hbm.at[pl.ds(e * A, A)],
                                      zf_v.at[s], sem_in.at[s]))

    def out_copies(e, s):
        cps = [pltpu.make_async_copy(
                   obs_v.at[s],
                   out_hbm.at[e * A + i, :, pl.ds(0, D)],
                   sem_out.at[s]) for i in range(A)]
        cps.append(pltpu.make_async_copy(
            zsh_v.at[s],
            out_hbm.at[pl.ds(e * A, A), :, pl.ds(D, NA)],
            sem_out.at[s]))
        return cps

    def start_in(e, s):
        for c in in_copies(e, s):
            c.start()

    def step(k, _):
        s = lax.rem(k, SLOTS)
        sn = lax.rem(k + 1, SLOTS)
        @pl.when(k >= SLOTS - 1)
        def _():
            for c in out_copies(env0, sn):
                c.wait()
        @pl.when(k + 1 < eperw)
        def _():
            start_in(env0 + k + 1, sn)
        for c in in_copies(env0, s):
            c.wait()
        for i in range(A):
            for j in range(A):
                zsh_v[s, i, j, :] = zf_v[s, i, pl.ds(j * NA, NA)]
        for c in out_copies(env0 + k, s):
            c.start()
        return 0

    start_in(env0, 0)
    lax.fori_loop(0, eperw, step, 0)
    for s in range(1, SLOTS):
        for c in out_copies(env0, s):
            c.wait()


def _sc_stage(obs_proc, zf, env_base):
    D = obs_proc.shape[1]
    ns = zf.shape[0]
    NA = zf.shape[1] // A
    eperw = ns // (A * NWORKERS)
    mesh = plsc.VectorSubcoreMesh(core_axis_name="c", subcore_axis_name="s")
    fn = functools.partial(
        pl.kernel,
        functools.partial(_sc_assemble_body, eperw, env_base),
        out_type=jax.ShapeDtypeStruct((ns, A, D + NA), jnp.float32),
        mesh=mesh,
        scratch_types=[
            pltpu.VMEM((SLOTS, A, D), jnp.float32),
            pltpu.VMEM((SLOTS, A, A * NA), jnp.float32),
            pltpu.VMEM((SLOTS, A, A, NA), jnp.float32),
            pltpu.SemaphoreType.DMA((SLOTS,)),
            pltpu.SemaphoreType.DMA((SLOTS,)),
        ],
    )()
    return fn(obs_proc, zf)


def kernel(h, policies, actions, obs_proc, edge_index,
           kW1, kb1, kW2, kb2, qW1, qb1, qW2, qb2):
    N, D = h.shape
    NA = policies.shape[1]
    HID = kW1.shape[1]
    OUT = kW2.shape[1]

    kb1r = kb1.reshape(1, HID); kb2r = kb2.reshape(1, OUT)
    qb1r = qb1.reshape(1, HID); qb2r = qb2.reshape(1, OUT)

    SLABS = 1
    ns = N // SLABS
    outs, ws = [], []
    for s in range(SLABS):
        zf, w2 = _tc_stage(h, policies, actions,
                           kW1, kb1r, kW2, kb2r, qW1, qb1r, qW2, qb2r,
                           ns, s * ns)
        outs.append(_sc_stage(obs_proc, zf, s * ns // A))
        ws.append(w2)
    out_final = jnp.concatenate(outs, axis=0)
    w_final = jnp.concatenate(ws, axis=0)
    return out_final, w_final.reshape(N, A, 1)
